# baseline (speedup 1.0000x reference)
import functools

import jax
import jax.numpy as jnp
from jax import lax
from jax.experimental import pallas as pl
from jax.experimental.pallas import tpu as pltpu
from jax.experimental.pallas import tpu_sc as plsc

B = 16384
NROWS = 1_000_000
D = 64
NC = 2
NS = 16
L = 16
N_TILE = B // NS
N_FIN = B // (NC * NS)
CHUNK = 128
NCHUNK = N_TILE // CHUNK
NFCHUNK = N_FIN // CHUNK
DUMMY = NROWS
TBL = NROWS + 8
ROUNDS = 4

_mesh = plsc.VectorSubcoreMesh(
    core_axis_name="c", subcore_axis_name="s", num_cores=NC
)


@functools.partial(
    pl.kernel,
    mesh=_mesh,
    compiler_params=pltpu.CompilerParams(use_tc_tiling_on_sc=False),
    scratch_types=[
        pltpu.VMEM_SHARED((TBL,), jnp.int32),
        pltpu.VMEM((NCHUNK, CHUNK), jnp.int32),
        pltpu.VMEM((NCHUNK, CHUNK), jnp.int32),
        pltpu.VMEM((NCHUNK, CHUNK), jnp.int32),
        pltpu.VMEM((NCHUNK, CHUNK), jnp.int32),
        pltpu.VMEM((NFCHUNK, CHUNK), jnp.int32),
        pltpu.VMEM((NFCHUNK, CHUNK), jnp.int32),
        pltpu.VMEM((N_FIN, D), jnp.float32),
        pltpu.SemaphoreType.DMA,
    ],
)
def _sc_scatter(out_ref, idx_hbm, upd_hbm, tbl, idx_v, pos_v, sidx_v,
                w_v, fidx_v, fw_v, frows_v, sem):
    c = lax.axis_index("c")
    s = lax.axis_index("s")
    base = s * N_TILE
    lane = lax.iota(jnp.int32, L)

    pltpu.sync_copy(idx_hbm.at[pl.ds(s * NCHUNK, NCHUNK)], idx_v)
    fin_base = c * (B // NC) + s * N_FIN
    pltpu.sync_copy(idx_hbm.at[pl.ds(fin_base // CHUNK, NFCHUNK)], fidx_v)
    for j in range(NCHUNK):
        for k in range(CHUNK // L):
            pos_v[j, pl.ds(k * L, L)] = base + (j * CHUNK + k * L) + lane

    def _scatter_pos(index_ref):
        cps = [pltpu.async_copy(pos_v.at[j], tbl.at[index_ref.at[j]], sem)
               for j in range(NCHUNK)]
        for c_ in cps:
            c_.wait()

    def _gather_w():
        cps = [pltpu.async_copy(tbl.at[idx_v.at[j]], w_v.at[j], sem)
               for j in range(NCHUNK)]
        for c_ in cps:
            c_.wait()

    _scatter_pos(idx_v)
    plsc.subcore_barrier()
    _gather_w()

    for _ in range(ROUNDS):
        for j in range(NCHUNK):
            for k in range(CHUNK // L):
                sl = pl.ds(k * L, L)
                p = pos_v[j, sl]
                w = w_v[j, sl]
                sidx_v[j, sl] = jnp.where(p > w, idx_v[j, sl], DUMMY)
        plsc.subcore_barrier()
        _scatter_pos(sidx_v)
        plsc.subcore_barrier()
        _gather_w()
    plsc.subcore_barrier()

    cps = [pltpu.async_copy(tbl.at[fidx_v.at[j]], fw_v.at[j], sem)
           for j in range(NFCHUNK)]
    for c_ in cps:
        c_.wait()
    cps = [pltpu.async_copy(upd_hbm.at[fw_v.at[j]],
                            frows_v.at[pl.ds(j * CHUNK, CHUNK)], sem)
           for j in range(NFCHUNK)]
    for c_ in cps:
        c_.wait()
    cps = [pltpu.async_copy(frows_v.at[pl.ds(j * CHUNK, CHUNK)],
                            out_ref.at[fidx_v.at[j]], sem)
           for j in range(NFCHUNK)]
    for c_ in cps:
        c_.wait()


def kernel(data, indices, updates):
    idx = indices.reshape(B).astype(jnp.int32).reshape(B // CHUNK, CHUNK)
    data_ref = jax.new_ref(data)
    _sc_scatter(data_ref, idx, updates)
    return jax.freeze(data_ref)
